# baseline (speedup 1.0000x reference)
import jax
import jax.numpy as jnp
from jax.experimental import pallas as pl
from jax.experimental.pallas import tpu as pltpu

CHANNELS = 768
TOPK = 384
TEMP = 1.0
C_BLK = 128
N_CBLK = CHANNELS // C_BLK
NB = 16
H = 56
W = 56
ROWS = NB * H * W
R_BLK = 1024
N_RBLK = ROWS // R_BLK


def _mask_kernel(logits_ref, m_ref, meta_ref):
    lg = logits_ref[0, :]
    col = lg[None, :]
    row = lg[:, None]
    i_idx = jax.lax.broadcasted_iota(jnp.int32, (CHANNELS, CHANNELS), 0)
    j_idx = jax.lax.broadcasted_iota(jnp.int32, (CHANNELS, CHANNELS), 1)
    beats = (col > row) | ((col == row) & (j_idx < i_idx))
    rank = jnp.sum(beats.astype(jnp.int32), axis=1)
    hard = (rank < TOPK).astype(jnp.float32)
    soft = jax.nn.sigmoid(lg / TEMP)
    m = (hard - soft) + soft
    m_ref[0, :] = m

    act = (jnp.sum(hard.reshape(N_CBLK, C_BLK), axis=1) > 0).astype(jnp.int32)
    a_col = act[None, :]
    ci = jax.lax.broadcasted_iota(jnp.int32, (N_CBLK, N_CBLK), 0)
    cj = jax.lax.broadcasted_iota(jnp.int32, (N_CBLK, N_CBLK), 1)
    inc = jnp.sum(jnp.where(cj <= ci, a_col, 0), axis=1)
    num_active = jnp.sum(act)
    c_lin = jax.lax.broadcasted_iota(jnp.int32, (1, N_CBLK), 1)[0]
    pos = jnp.where(act == 1, inc - 1, num_active + c_lin - inc)
    perm = jnp.sum(jnp.where(pos[None, :] == ci, cj, 0), axis=1)
    last_active = jnp.sum(jnp.where(c_lin == num_active - 1, perm, 0))
    zidx = jnp.where(c_lin < num_active, perm, last_active)

    c_sub = jax.lax.broadcasted_iota(jnp.int32, (N_CBLK, 128), 0)
    lane2 = jax.lax.broadcasted_iota(jnp.int32, (N_CBLK, 128), 1)
    meta = (jnp.sum(jnp.where(lane2 == c_sub, zidx[:, None], 0), axis=0)
            + jnp.sum(jnp.where(lane2 == c_sub + 8, perm[:, None], 0), axis=0))
    lane = jax.lax.broadcasted_iota(jnp.int32, (1, 128), 1)
    meta = meta + jnp.where(lane[0] == 7, num_active, 0)
    meta_ref[0, :] = meta


def _gate_kernel(meta_ref, z_ref, m_ref, out_ref):
    del meta_ref
    out_ref[...] = z_ref[...] * m_ref[0][None, :]


def kernel(z, logits):
    zt = z.transpose(0, 2, 3, 1).reshape(ROWS, CHANNELS)
    m_out, meta = pl.pallas_call(
        _mask_kernel,
        out_shape=(
            jax.ShapeDtypeStruct((1, CHANNELS), jnp.float32),
            jax.ShapeDtypeStruct((1, 128), jnp.int32),
        ),
    )(logits.reshape(1, CHANNELS))

    def z_map(p, r, meta):
        return (jnp.where(p < meta[0, 7], r, N_RBLK - 1), meta[0, p])

    def m_map(p, r, meta):
        return (0, meta[0, 8 + p])

    def out_map(p, r, meta):
        return (r, meta[0, 8 + p])

    grid_spec = pltpu.PrefetchScalarGridSpec(
        num_scalar_prefetch=1,
        grid=(N_CBLK, N_RBLK),
        in_specs=[
            pl.BlockSpec((R_BLK, C_BLK), z_map),
            pl.BlockSpec((1, C_BLK), m_map),
        ],
        out_specs=pl.BlockSpec((R_BLK, C_BLK), out_map),
    )
    out = pl.pallas_call(
        _gate_kernel,
        grid_spec=grid_spec,
        out_shape=jax.ShapeDtypeStruct((ROWS, CHANNELS), jnp.float32),
    )(meta, zt, m_out)
    return out.reshape(NB, H, W, CHANNELS).transpose(0, 3, 1, 2)


# speedup vs baseline: 2.1403x; 2.1403x over previous
import jax
import jax.numpy as jnp
from jax.experimental import pallas as pl
from jax.experimental.pallas import tpu as pltpu

CHANNELS = 768
TOPK = 384
TEMP = 1.0
C_BLK = 128
N_CBLK = CHANNELS // C_BLK
NB = 16
H = 56
W = 56
ROWS = NB * H * W
R_BLK = 1024
N_RBLK = ROWS // R_BLK


def _mask_kernel(logits_ref, m_ref, meta_ref):
    lg = logits_ref[0, :]
    col = lg[None, :]
    row = lg[:, None]
    i_idx = jax.lax.broadcasted_iota(jnp.int32, (CHANNELS, CHANNELS), 0)
    j_idx = jax.lax.broadcasted_iota(jnp.int32, (CHANNELS, CHANNELS), 1)
    beats = (col > row) | ((col == row) & (j_idx < i_idx))
    rank = jnp.sum(beats.astype(jnp.int32), axis=1)
    hard = (rank < TOPK).astype(jnp.float32)
    soft = jax.nn.sigmoid(lg / TEMP)
    m = (hard - soft) + soft
    m_ref[0, :] = m

    wact = (jnp.sum(hard.reshape(2, CHANNELS // 2), axis=1) > 0).astype(jnp.int32)
    lane = jax.lax.broadcasted_iota(jnp.int32, (1, 128), 1)[0]
    meta = (jnp.where(lane == 0, wact[0], 0)
            + jnp.where(lane == 1, wact[1], 0))
    meta_ref[0, :] = meta


HALF = CHANNELS // 2


def _gate_kernel(meta_ref, z0_ref, z1_ref, m_ref, out_ref):
    del meta_ref
    out_ref[:, :HALF] = z0_ref[...] * m_ref[0, :HALF][None, :]
    out_ref[:, HALF:] = z1_ref[...] * m_ref[0, HALF:][None, :]


def kernel(z, logits):
    zt = z.transpose(0, 2, 3, 1).reshape(ROWS, CHANNELS)
    m_out, meta = pl.pallas_call(
        _mask_kernel,
        out_shape=(
            jax.ShapeDtypeStruct((1, CHANNELS), jnp.float32),
            jax.ShapeDtypeStruct((1, 128), jnp.int32),
        ),
    )(logits.reshape(1, CHANNELS))

    def z0_map(r, meta):
        return (jnp.where(meta[0, 0] > 0, r, N_RBLK - 1), 0)

    def z1_map(r, meta):
        return (jnp.where(meta[0, 1] > 0, r, N_RBLK - 1), 1)

    grid_spec = pltpu.PrefetchScalarGridSpec(
        num_scalar_prefetch=1,
        grid=(N_RBLK,),
        in_specs=[
            pl.BlockSpec((R_BLK, HALF), z0_map),
            pl.BlockSpec((R_BLK, HALF), z1_map),
            pl.BlockSpec((1, CHANNELS), lambda r, meta: (0, 0)),
        ],
        out_specs=pl.BlockSpec((R_BLK, CHANNELS), lambda r, meta: (r, 0)),
    )
    out = pl.pallas_call(
        _gate_kernel,
        grid_spec=grid_spec,
        out_shape=jax.ShapeDtypeStruct((ROWS, CHANNELS), jnp.float32),
    )(meta, zt, zt, m_out)
    return out.reshape(NB, H, W, CHANNELS).transpose(0, 3, 1, 2)


# speedup vs baseline: 2.2465x; 1.0496x over previous
import jax
import jax.numpy as jnp
from jax.experimental import pallas as pl
from jax.experimental.pallas import tpu as pltpu

CHANNELS = 768
TOPK = 384
TEMP = 1.0
C_BLK = 128
N_CBLK = CHANNELS // C_BLK
NB = 16
H = 56
W = 56
ROWS = NB * H * W
R_BLK = 1792
N_RBLK = ROWS // R_BLK


def _mask_kernel(logits_ref, m_ref, meta_ref):
    lg = logits_ref[0, :]
    col = lg[None, :]
    row = lg[:, None]
    i_idx = jax.lax.broadcasted_iota(jnp.int32, (CHANNELS, CHANNELS), 0)
    j_idx = jax.lax.broadcasted_iota(jnp.int32, (CHANNELS, CHANNELS), 1)
    beats = (col > row) | ((col == row) & (j_idx < i_idx))
    rank = jnp.sum(beats.astype(jnp.int32), axis=1)
    hard = (rank < TOPK).astype(jnp.float32)
    soft = jax.nn.sigmoid(lg / TEMP)
    m = (hard - soft) + soft
    m_ref[0, :] = m

    wact = (jnp.sum(hard.reshape(2, CHANNELS // 2), axis=1) > 0).astype(jnp.int32)
    lane = jax.lax.broadcasted_iota(jnp.int32, (1, 128), 1)[0]
    meta = (jnp.where(lane == 0, wact[0], 0)
            + jnp.where(lane == 1, wact[1], 0))
    meta_ref[0, :] = meta


HALF = CHANNELS // 2


def _gate_kernel(meta_ref, z0_ref, z1_ref, m_ref, out_ref):
    del meta_ref
    out_ref[:, :HALF] = z0_ref[...] * m_ref[0, :HALF][None, :]
    out_ref[:, HALF:] = z1_ref[...] * m_ref[0, HALF:][None, :]


def kernel(z, logits):
    zt = z.transpose(0, 2, 3, 1).reshape(ROWS, CHANNELS)
    m_out, meta = pl.pallas_call(
        _mask_kernel,
        out_shape=(
            jax.ShapeDtypeStruct((1, CHANNELS), jnp.float32),
            jax.ShapeDtypeStruct((1, 128), jnp.int32),
        ),
    )(logits.reshape(1, CHANNELS))

    def z0_map(r, meta):
        return (jnp.where(meta[0, 0] > 0, r, N_RBLK - 1), 0)

    def z1_map(r, meta):
        return (jnp.where(meta[0, 1] > 0, r, N_RBLK - 1), 1)

    grid_spec = pltpu.PrefetchScalarGridSpec(
        num_scalar_prefetch=1,
        grid=(N_RBLK,),
        in_specs=[
            pl.BlockSpec((R_BLK, HALF), z0_map),
            pl.BlockSpec((R_BLK, HALF), z1_map),
            pl.BlockSpec((1, CHANNELS), lambda r, meta: (0, 0)),
        ],
        out_specs=pl.BlockSpec((R_BLK, CHANNELS), lambda r, meta: (r, 0)),
    )
    out = pl.pallas_call(
        _gate_kernel,
        grid_spec=grid_spec,
        out_shape=jax.ShapeDtypeStruct((ROWS, CHANNELS), jnp.float32),
    )(meta, zt, zt, m_out)
    return out.reshape(NB, H, W, CHANNELS).transpose(0, 3, 1, 2)


# speedup vs baseline: 2.3113x; 1.0288x over previous
import jax
import jax.numpy as jnp
from jax.experimental import pallas as pl
from jax.experimental.pallas import tpu as pltpu

CHANNELS = 768
TOPK = 384
TEMP = 1.0
C_BLK = 128
N_CBLK = CHANNELS // C_BLK
NB = 16
H = 56
W = 56
ROWS = NB * H * W
R_BLK = 3584
N_RBLK = ROWS // R_BLK


def _mask_kernel(logits_ref, m_ref, meta_ref):
    lg = logits_ref[0, :]
    col = lg[None, :]
    row = lg[:, None]
    i_idx = jax.lax.broadcasted_iota(jnp.int32, (CHANNELS, CHANNELS), 0)
    j_idx = jax.lax.broadcasted_iota(jnp.int32, (CHANNELS, CHANNELS), 1)
    beats = (col > row) | ((col == row) & (j_idx < i_idx))
    rank = jnp.sum(beats.astype(jnp.int32), axis=1)
    hard = (rank < TOPK).astype(jnp.float32)
    soft = jax.nn.sigmoid(lg / TEMP)
    m = (hard - soft) + soft
    m_ref[0, :] = m

    wact = (jnp.sum(hard.reshape(2, CHANNELS // 2), axis=1) > 0).astype(jnp.int32)
    lane = jax.lax.broadcasted_iota(jnp.int32, (1, 128), 1)[0]
    meta = (jnp.where(lane == 0, wact[0], 0)
            + jnp.where(lane == 1, wact[1], 0))
    meta_ref[0, :] = meta


HALF = CHANNELS // 2


def _gate_kernel(meta_ref, z0_ref, z1_ref, m_ref, out_ref):
    del meta_ref
    out_ref[:, :HALF] = z0_ref[...] * m_ref[0, :HALF][None, :]
    out_ref[:, HALF:] = z1_ref[...] * m_ref[0, HALF:][None, :]


def kernel(z, logits):
    zt = z.transpose(0, 2, 3, 1).reshape(ROWS, CHANNELS)
    m_out, meta = pl.pallas_call(
        _mask_kernel,
        out_shape=(
            jax.ShapeDtypeStruct((1, CHANNELS), jnp.float32),
            jax.ShapeDtypeStruct((1, 128), jnp.int32),
        ),
    )(logits.reshape(1, CHANNELS))

    def z0_map(r, meta):
        return (jnp.where(meta[0, 0] > 0, r, N_RBLK - 1), 0)

    def z1_map(r, meta):
        return (jnp.where(meta[0, 1] > 0, r, N_RBLK - 1), 1)

    grid_spec = pltpu.PrefetchScalarGridSpec(
        num_scalar_prefetch=1,
        grid=(N_RBLK,),
        in_specs=[
            pl.BlockSpec((R_BLK, HALF), z0_map),
            pl.BlockSpec((R_BLK, HALF), z1_map),
            pl.BlockSpec((1, CHANNELS), lambda r, meta: (0, 0)),
        ],
        out_specs=pl.BlockSpec((R_BLK, CHANNELS), lambda r, meta: (r, 0)),
    )
    out = pl.pallas_call(
        _gate_kernel,
        grid_spec=grid_spec,
        out_shape=jax.ShapeDtypeStruct((ROWS, CHANNELS), jnp.float32),
    )(meta, zt, zt, m_out)
    return out.reshape(NB, H, W, CHANNELS).transpose(0, 3, 1, 2)
